# baseline (speedup 1.0000x reference)
import functools

import jax
import jax.numpy as jnp
from jax import lax
from jax.experimental import pallas as pl
from jax.experimental.pallas import tpu as pltpu
from jax.experimental.pallas import tpu_sc as plsc

N = 160000
D = 256
NH = 4
HP = 16
NSEG = 10000
EPS = 1e-5

LIN = N * HP // 128
OLIN = N // 128

TB = 3200
TB2L = 2048

NW = 32
CH = N // NW
CW = 125
NCH = CH // CW
NSEGP = 10240
SEG_STRIPE = NSEGP // 16


def _pack_lanes(v, groups):
    r, k = v.shape
    v3 = v.reshape(r // groups, groups, k)
    return jnp.concatenate([v3[:, u, :] for u in range(groups)], axis=1)


def _tc1_body(x_ref, w1_ref, w2_ref, aug_ref, b1_ref, b2_ref,
              l8_ref, l9_ref, bbp_ref, maskp_ref, o_ref):
    xb = x_ref[...].astype(jnp.bfloat16)
    acc1 = jnp.dot(xb, w1_ref[...], preferred_element_type=jnp.float32)
    h1b = jnp.maximum(acc1.astype(jnp.bfloat16) + b1_ref[...], 0)
    acc2 = jnp.dot(h1b, w2_ref[...], preferred_element_type=jnp.float32)
    h2b = jnp.maximum(acc2.astype(jnp.bfloat16) + b2_ref[...], 0)
    lhs = jnp.concatenate([h2b, h2b * h2b], axis=1)
    tm = jnp.dot(lhs, aug_ref[...], preferred_element_type=jnp.float32)
    tp = _pack_lanes(tm, 128 // HP)
    mu = tp * l8_ref[...] * (1.0 / D)
    s2l = jnp.roll(tp * l9_ref[...], -1, axis=1) * (1.0 / D)
    v = s2l - mu * mu
    v = v + jnp.roll(v, 1, axis=1)
    v = v + jnp.roll(v, 2, axis=1)
    v = v + jnp.roll(v, 4, axis=1)
    v = v + jnp.roll(v, -8, axis=1)
    inv = jax.lax.rsqrt(v + EPS)
    raw = inv * tp + bbp_ref[...]
    o_ref[...] = jnp.exp(jax.nn.sigmoid(raw)) * maskp_ref[...]


def _tc1(x, w1t, w2t, aug, b1r, b2r, l8, l9, bbp, maskp):
    grid = (N // TB,)
    full = lambda shape: pl.BlockSpec(shape, lambda i: (0, 0))
    return pl.pallas_call(
        _tc1_body,
        grid=grid,
        in_specs=[
            pl.BlockSpec((TB, D), lambda i: (i, 0)),
            full((D, D)), full((D, D)), full((2 * D, HP)),
            full((1, D)), full((1, D)),
            full((1, 128)), full((1, 128)), full((1, 128)), full((1, 128)),
        ],
        out_specs=pl.BlockSpec((TB * HP // 128, 128), lambda i: (i, 0)),
        out_shape=jax.ShapeDtypeStruct((LIN, 128), jnp.float32),
    )(x, w1t, w2t, aug, b1r, b2r, l8, l9, bbp, maskp)


def _sc_mesh():
    return plsc.VectorSubcoreMesh(core_axis_name="c", subcore_axis_name="s")


@jax.jit
def _sc_k1(e, row3, zeros):
    @functools.partial(
        pl.kernel,
        out_type=jax.ShapeDtypeStruct((2, NSEGP, HP), jnp.float32),
        mesh=_sc_mesh(),
        compiler_params=pltpu.CompilerParams(use_tc_tiling_on_sc=False),
        scratch_types=[
            pltpu.VMEM((CH, HP), jnp.float32),
            pltpu.VMEM((NCH, CW), jnp.int32),
            pltpu.VMEM_SHARED((NSEGP, HP), jnp.float32),
            pltpu.SemaphoreType.DMA,
        ],
    )
    def k(e_hbm, row_hbm, z_hbm, p_hbm, e_v, row_v, acc_sh, sem):
        c = lax.axis_index("c")
        s = lax.axis_index("s")
        wid = s * 2 + c

        @pl.when(s == 0)
        def _():
            pltpu.sync_copy(z_hbm, acc_sh)

        pltpu.async_copy(e_hbm.at[pl.ds(wid * CH, CH)], e_v, sem).wait()
        pltpu.async_copy(row_hbm.at[wid], row_v, sem).wait()
        plsc.subcore_barrier()

        @pl.loop(0, NCH)
        def _(j):
            pltpu.sync_copy(e_v.at[pl.ds(j * CW, CW)],
                            acc_sh.at[row_v.at[j]], add=True)

        plsc.subcore_barrier()
        pltpu.sync_copy(acc_sh.at[pl.ds(s * SEG_STRIPE, SEG_STRIPE)],
                        p_hbm.at[c].at[pl.ds(s * SEG_STRIPE, SEG_STRIPE)])

    return k(e, row3, zeros)


@jax.jit
def _sc_k2(recip, row3):
    @functools.partial(
        pl.kernel,
        out_type=jax.ShapeDtypeStruct((N, HP), jnp.float32),
        mesh=_sc_mesh(),
        compiler_params=pltpu.CompilerParams(use_tc_tiling_on_sc=False),
        scratch_types=[
            pltpu.VMEM((CH, HP), jnp.float32),
            pltpu.VMEM((NCH, CW), jnp.int32),
            pltpu.VMEM_SHARED((NSEGP, HP), jnp.float32),
            pltpu.SemaphoreType.DMA,
        ],
    )
    def k(recip_hbm, row_hbm, r_hbm, g_v, row_v, recip_sh, sem):
        c = lax.axis_index("c")
        s = lax.axis_index("s")
        wid = s * 2 + c

        @pl.when(s == 0)
        def _():
            pltpu.sync_copy(recip_hbm, recip_sh)

        pltpu.async_copy(row_hbm.at[wid], row_v, sem).wait()
        plsc.subcore_barrier()

        @pl.loop(0, NCH)
        def _(j):
            pltpu.sync_copy(recip_sh.at[row_v.at[j]],
                            g_v.at[pl.ds(j * CW, CW)])

        pltpu.sync_copy(g_v, r_hbm.at[pl.ds(wid * CH, CH)])

    return k(recip, row3)


PHALF = NSEGP * HP // 128


def _tcmid_body(p_ref, o_ref):
    acc = p_ref[:PHALF, :] + p_ref[PHALF:, :]
    o_ref[...] = jnp.where(acc > 0.0, 1.0 / acc, 0.0)


def _tcmid(p_lin):
    return pl.pallas_call(
        _tcmid_body,
        in_specs=[pl.BlockSpec((2 * PHALF, 128), lambda: (0, 0))],
        out_specs=pl.BlockSpec((PHALF, 128), lambda: (0, 0)),
        out_shape=jax.ShapeDtypeStruct((PHALF, 128), jnp.float32),
    )(p_lin)


def _tc2_body(e_ref, r_ref, g_ref, o_ref):
    prod = e_ref[...] * r_ref[...]
    sums = 0.25 * jnp.dot(prod, g_ref[...],
                          preferred_element_type=jnp.float32)
    o_ref[...] = _pack_lanes(sums, HP)


def _tc2(e_lin, r_lin, g):
    grid = ((LIN + TB2L - 1) // TB2L,)
    return pl.pallas_call(
        _tc2_body,
        grid=grid,
        in_specs=[pl.BlockSpec((TB2L, 128), lambda i: (i, 0)),
                  pl.BlockSpec((TB2L, 128), lambda i: (i, 0)),
                  pl.BlockSpec((128, 128 // HP), lambda i: (0, 0))],
        out_specs=pl.BlockSpec((TB2L // HP, 128), lambda i: (i, 0)),
        out_shape=jax.ShapeDtypeStruct((OLIN, 128), jnp.float32),
    )(e_lin, r_lin, g)


def kernel(x, row, alpha, W1, b1, W2, b2, ln_g, ln_b, Wa, ba):
    b1_eff = (b1 + alpha[0, 0] * W1[:, D]).reshape(1, D).astype(jnp.bfloat16)
    w1t = W1[:, :D].T.astype(jnp.bfloat16)
    w2t = W2.T.astype(jnp.bfloat16)
    wa_g = Wa * ln_g[None, :]
    swa = jnp.sum(wa_g, axis=1)
    aug = (jnp.zeros((2 * D, HP), jnp.float32)
           .at[:D, :NH].set(wa_g.T - swa[None, :] * (1.0 / D))
           .at[:D, 8].set(1.0)
           .at[D:, 9].set(1.0)).astype(jnp.bfloat16)
    bb = ba + Wa @ ln_b
    lane = jnp.arange(128)
    l8 = (lane % HP == 8).astype(jnp.float32).reshape(1, 128)
    l9 = (lane % HP == 9).astype(jnp.float32).reshape(1, 128)
    bbp = jnp.where(lane % HP < NH, bb[lane % NH], 0.0).reshape(1, 128)
    maskp = (lane % HP < NH).astype(jnp.float32).reshape(1, 128)
    row3 = row.reshape(NW, NCH, CW)
    zeros = jnp.zeros((NSEGP, HP), jnp.float32)
    g = (jnp.arange(128)[:, None] // HP ==
         jnp.arange(128 // HP)[None, :]).astype(jnp.float32)

    e_lin = _tc1(x, w1t, w2t, aug, b1_eff,
                 b2.reshape(1, D).astype(jnp.bfloat16), l8, l9, bbp, maskp)
    e8 = e_lin.reshape(N, HP)
    partials = _sc_k1(e8, row3, zeros)
    recip_lin = _tcmid(partials.reshape(2 * PHALF, 128))
    r8 = _sc_k2(recip_lin.reshape(NSEGP, HP), row3)
    out_lin = _tc2(e_lin, r8.reshape(LIN, 128), g)
    return out_lin.reshape(N, 1)


# speedup vs baseline: 1.1320x; 1.1320x over previous
import functools

import jax
import jax.numpy as jnp
from jax import lax
from jax.experimental import pallas as pl
from jax.experimental.pallas import tpu as pltpu
from jax.experimental.pallas import tpu_sc as plsc

N = 160000
D = 256
NH = 4
HP = 16
NSEG = 10000
EPS = 1e-5

LIN = N * HP // 128
OLIN = N // 128

TB = 3200
TB2L = 2048

NW = 32
CH = N // NW
CW = 125
NCH = CH // CW
NSEGP = 10240
SEG_STRIPE = NSEGP // 16


def _pack_lanes(v, groups):
    r, k = v.shape
    v3 = v.reshape(r // groups, groups, k)
    return jnp.concatenate([v3[:, u, :] for u in range(groups)], axis=1)


def _tc1_body(x_ref, w1_ref, w2_ref, aug_ref, b1_ref, b2_ref,
              l8_ref, l9_ref, bbp_ref, maskp_ref, o_ref):
    HB = TB // 8
    HO = HB * HP // 128
    for half in range(8):
        xb = x_ref[half * HB:(half + 1) * HB, :].astype(jnp.bfloat16)
        acc1 = jnp.dot(xb, w1_ref[...], preferred_element_type=jnp.float32)
        h1b = jnp.maximum(acc1.astype(jnp.bfloat16) + b1_ref[...], 0)
        acc2 = jnp.dot(h1b, w2_ref[...], preferred_element_type=jnp.float32)
        h2b = jnp.maximum(acc2.astype(jnp.bfloat16) + b2_ref[...], 0)
        lhs = jnp.concatenate([h2b, h2b * h2b], axis=1)
        tm = jnp.dot(lhs, aug_ref[...], preferred_element_type=jnp.float32)
        tp = _pack_lanes(tm, 128 // HP)
        mu = tp * l8_ref[...] * (1.0 / D)
        s2l = jnp.roll(tp * l9_ref[...], -1, axis=1) * (1.0 / D)
        v = s2l - mu * mu
        v = v + jnp.roll(v, 1, axis=1)
        v = v + jnp.roll(v, 2, axis=1)
        v = v + jnp.roll(v, 4, axis=1)
        v = v + jnp.roll(v, -8, axis=1)
        inv = jax.lax.rsqrt(v + EPS)
        raw = inv * tp + bbp_ref[...]
        e = jnp.exp(jax.nn.sigmoid(raw)) * maskp_ref[...]
        o_ref[half * HO:(half + 1) * HO, :] = e


def _tc1(x, w1t, w2t, aug, b1r, b2r, l8, l9, bbp, maskp):
    grid = (N // TB,)
    full = lambda shape: pl.BlockSpec(shape, lambda i: (0, 0))
    return pl.pallas_call(
        _tc1_body,
        grid=grid,
        in_specs=[
            pl.BlockSpec((TB, D), lambda i: (i, 0)),
            full((D, D)), full((D, D)), full((2 * D, HP)),
            full((1, D)), full((1, D)),
            full((1, 128)), full((1, 128)), full((1, 128)), full((1, 128)),
        ],
        out_specs=pl.BlockSpec((TB * HP // 128, 128), lambda i: (i, 0)),
        out_shape=jax.ShapeDtypeStruct((LIN, 128), jnp.float32),
    )(x, w1t, w2t, aug, b1r, b2r, l8, l9, bbp, maskp)


def _sc_mesh():
    return plsc.VectorSubcoreMesh(core_axis_name="c", subcore_axis_name="s")


@jax.jit
def _sc_k1(e, row3, zeros):
    @functools.partial(
        pl.kernel,
        out_type=jax.ShapeDtypeStruct((2, NSEGP, HP), jnp.float32),
        mesh=_sc_mesh(),
        compiler_params=pltpu.CompilerParams(use_tc_tiling_on_sc=False),
        scratch_types=[
            pltpu.VMEM((CH, HP), jnp.float32),
            pltpu.VMEM((NCH, CW), jnp.int32),
            pltpu.VMEM_SHARED((NSEGP, HP), jnp.float32),
            pltpu.SemaphoreType.DMA,
        ],
    )
    def k(e_hbm, row_hbm, z_hbm, p_hbm, e_v, row_v, acc_sh, sem):
        c = lax.axis_index("c")
        s = lax.axis_index("s")
        wid = s * 2 + c

        @pl.when(s == 0)
        def _():
            pltpu.sync_copy(z_hbm, acc_sh)

        pltpu.async_copy(e_hbm.at[pl.ds(wid * CH, CH)], e_v, sem).wait()
        pltpu.async_copy(row_hbm.at[wid], row_v, sem).wait()
        plsc.subcore_barrier()

        @pl.loop(0, NCH)
        def _(j):
            pltpu.sync_copy(e_v.at[pl.ds(j * CW, CW)],
                            acc_sh.at[row_v.at[j]], add=True)

        plsc.subcore_barrier()
        pltpu.sync_copy(acc_sh.at[pl.ds(s * SEG_STRIPE, SEG_STRIPE)],
                        p_hbm.at[c].at[pl.ds(s * SEG_STRIPE, SEG_STRIPE)])

    return k(e, row3, zeros)


@jax.jit
def _sc_k2(recip, row3):
    @functools.partial(
        pl.kernel,
        out_type=jax.ShapeDtypeStruct((N, HP), jnp.float32),
        mesh=_sc_mesh(),
        compiler_params=pltpu.CompilerParams(use_tc_tiling_on_sc=False),
        scratch_types=[
            pltpu.VMEM((CH, HP), jnp.float32),
            pltpu.VMEM((NCH, CW), jnp.int32),
            pltpu.VMEM_SHARED((NSEGP, HP), jnp.float32),
            pltpu.SemaphoreType.DMA,
        ],
    )
    def k(recip_hbm, row_hbm, r_hbm, g_v, row_v, recip_sh, sem):
        c = lax.axis_index("c")
        s = lax.axis_index("s")
        wid = s * 2 + c

        @pl.when(s == 0)
        def _():
            pltpu.sync_copy(recip_hbm, recip_sh)

        pltpu.async_copy(row_hbm.at[wid], row_v, sem).wait()
        plsc.subcore_barrier()

        @pl.loop(0, NCH)
        def _(j):
            pltpu.sync_copy(recip_sh.at[row_v.at[j]],
                            g_v.at[pl.ds(j * CW, CW)])

        pltpu.sync_copy(g_v, r_hbm.at[pl.ds(wid * CH, CH)])

    return k(recip, row3)


PHALF = NSEGP * HP // 128


def _tcmid_body(p_ref, o_ref):
    acc = p_ref[:PHALF, :] + p_ref[PHALF:, :]
    o_ref[...] = jnp.where(acc > 0.0, 1.0 / acc, 0.0)


def _tcmid(p_lin):
    return pl.pallas_call(
        _tcmid_body,
        in_specs=[pl.BlockSpec((2 * PHALF, 128), lambda: (0, 0))],
        out_specs=pl.BlockSpec((PHALF, 128), lambda: (0, 0)),
        out_shape=jax.ShapeDtypeStruct((PHALF, 128), jnp.float32),
    )(p_lin)


def _tc2_body(e_ref, r_ref, g_ref, o_ref):
    prod = e_ref[...] * r_ref[...]
    sums = 0.25 * jnp.dot(prod, g_ref[...],
                          preferred_element_type=jnp.float32)
    o_ref[...] = _pack_lanes(sums, HP)


def _tc2(e_lin, r_lin, g):
    grid = ((LIN + TB2L - 1) // TB2L,)
    return pl.pallas_call(
        _tc2_body,
        grid=grid,
        in_specs=[pl.BlockSpec((TB2L, 128), lambda i: (i, 0)),
                  pl.BlockSpec((TB2L, 128), lambda i: (i, 0)),
                  pl.BlockSpec((128, 128 // HP), lambda i: (0, 0))],
        out_specs=pl.BlockSpec((TB2L // HP, 128), lambda i: (i, 0)),
        out_shape=jax.ShapeDtypeStruct((OLIN, 128), jnp.float32),
    )(e_lin, r_lin, g)


def kernel(x, row, alpha, W1, b1, W2, b2, ln_g, ln_b, Wa, ba):
    b1_eff = (b1 + alpha[0, 0] * W1[:, D]).reshape(1, D).astype(jnp.bfloat16)
    w1t = W1[:, :D].T.astype(jnp.bfloat16)
    w2t = W2.T.astype(jnp.bfloat16)
    wa_g = Wa * ln_g[None, :]
    swa = jnp.sum(wa_g, axis=1)
    aug = (jnp.zeros((2 * D, HP), jnp.float32)
           .at[:D, :NH].set(wa_g.T - swa[None, :] * (1.0 / D))
           .at[:D, 8].set(1.0)
           .at[D:, 9].set(1.0)).astype(jnp.bfloat16)
    bb = ba + Wa @ ln_b
    lane = jnp.arange(128)
    l8 = (lane % HP == 8).astype(jnp.float32).reshape(1, 128)
    l9 = (lane % HP == 9).astype(jnp.float32).reshape(1, 128)
    bbp = jnp.where(lane % HP < NH, bb[lane % NH], 0.0).reshape(1, 128)
    maskp = (lane % HP < NH).astype(jnp.float32).reshape(1, 128)
    row3 = row.reshape(NW, NCH, CW)
    zeros = jnp.zeros((NSEGP, HP), jnp.float32)
    g = (jnp.arange(128)[:, None] // HP ==
         jnp.arange(128 // HP)[None, :]).astype(jnp.float32)

    e_lin = _tc1(x, w1t, w2t, aug, b1_eff,
                 b2.reshape(1, D).astype(jnp.bfloat16), l8, l9, bbp, maskp)
    e8 = e_lin.reshape(N, HP)
    partials = _sc_k1(e8, row3, zeros)
    recip_lin = _tcmid(partials.reshape(2 * PHALF, 128))
    r8 = _sc_k2(recip_lin.reshape(NSEGP, HP), row3)
    out_lin = _tc2(e_lin, r8.reshape(LIN, 128), g)
    return out_lin.reshape(N, 1)


# speedup vs baseline: 1.2472x; 1.1017x over previous
import functools

import jax
import jax.numpy as jnp
from jax import lax
from jax.experimental import pallas as pl
from jax.experimental.pallas import tpu as pltpu
from jax.experimental.pallas import tpu_sc as plsc

N = 160000
D = 256
NH = 4
HP = 16
NSEG = 10000
EPS = 1e-5

LIN = N * HP // 128
OLIN = N // 128

TB = 6400
TB2L = 2048

NW = 32
CH = N // NW
CW = 125
NCH = CH // CW
NSEGP = 10240
SEG_STRIPE = NSEGP // 16


def _pack_lanes(v, groups):
    r, k = v.shape
    v3 = v.reshape(r // groups, groups, k)
    return jnp.concatenate([v3[:, u, :] for u in range(groups)], axis=1)


def _tc1_body(x_ref, w1_ref, w2_ref, aug_ref, b1_ref, b2_ref,
              l8_ref, l9_ref, bbp_ref, maskp_ref, o_ref):
    HB = TB // 16
    HO = HB * HP // 128
    for half in range(16):
        xb = x_ref[half * HB:(half + 1) * HB, :].astype(jnp.bfloat16)
        acc1 = jnp.dot(xb, w1_ref[...], preferred_element_type=jnp.float32)
        h1b = jnp.maximum(acc1.astype(jnp.bfloat16) + b1_ref[...], 0)
        acc2 = jnp.dot(h1b, w2_ref[...], preferred_element_type=jnp.float32)
        h2b = jnp.maximum(acc2.astype(jnp.bfloat16) + b2_ref[...], 0)
        lhs = jnp.concatenate([h2b, h2b * h2b], axis=1)
        tm = jnp.dot(lhs, aug_ref[...], preferred_element_type=jnp.float32)
        tp = _pack_lanes(tm, 128 // HP)
        mu = tp * l8_ref[...] * (1.0 / D)
        s2l = jnp.roll(tp * l9_ref[...], -1, axis=1) * (1.0 / D)
        v = s2l - mu * mu
        v = v + jnp.roll(v, 1, axis=1)
        v = v + jnp.roll(v, 2, axis=1)
        v = v + jnp.roll(v, 4, axis=1)
        v = v + jnp.roll(v, -8, axis=1)
        inv = jax.lax.rsqrt(v + EPS)
        raw = inv * tp + bbp_ref[...]
        e = jnp.exp(jax.nn.sigmoid(raw)) * maskp_ref[...]
        o_ref[half * HO:(half + 1) * HO, :] = e


def _tc1(x, w1t, w2t, aug, b1r, b2r, l8, l9, bbp, maskp):
    grid = (N // TB,)
    full = lambda shape: pl.BlockSpec(shape, lambda i: (0, 0))
    return pl.pallas_call(
        _tc1_body,
        grid=grid,
        in_specs=[
            pl.BlockSpec((TB, D), lambda i: (i, 0)),
            full((D, D)), full((D, D)), full((2 * D, HP)),
            full((1, D)), full((1, D)),
            full((1, 128)), full((1, 128)), full((1, 128)), full((1, 128)),
        ],
        out_specs=pl.BlockSpec((TB * HP // 128, 128), lambda i: (i, 0)),
        out_shape=jax.ShapeDtypeStruct((LIN, 128), jnp.float32),
    )(x, w1t, w2t, aug, b1r, b2r, l8, l9, bbp, maskp)


def _sc_mesh():
    return plsc.VectorSubcoreMesh(core_axis_name="c", subcore_axis_name="s")


@jax.jit
def _sc_k1(e, row3, zeros):
    @functools.partial(
        pl.kernel,
        out_type=jax.ShapeDtypeStruct((2, NSEGP, HP), jnp.float32),
        mesh=_sc_mesh(),
        compiler_params=pltpu.CompilerParams(use_tc_tiling_on_sc=False),
        scratch_types=[
            pltpu.VMEM((CH, HP), jnp.float32),
            pltpu.VMEM((NCH, CW), jnp.int32),
            pltpu.VMEM_SHARED((NSEGP, HP), jnp.float32),
            pltpu.SemaphoreType.DMA,
        ],
    )
    def k(e_hbm, row_hbm, z_hbm, p_hbm, e_v, row_v, acc_sh, sem):
        c = lax.axis_index("c")
        s = lax.axis_index("s")
        wid = s * 2 + c

        @pl.when(s == 0)
        def _():
            pltpu.sync_copy(z_hbm, acc_sh)

        pltpu.async_copy(e_hbm.at[pl.ds(wid * CH, CH)], e_v, sem).wait()
        pltpu.async_copy(row_hbm.at[wid], row_v, sem).wait()
        plsc.subcore_barrier()

        @pl.loop(0, NCH)
        def _(j):
            pltpu.sync_copy(e_v.at[pl.ds(j * CW, CW)],
                            acc_sh.at[row_v.at[j]], add=True)

        plsc.subcore_barrier()
        pltpu.sync_copy(acc_sh.at[pl.ds(s * SEG_STRIPE, SEG_STRIPE)],
                        p_hbm.at[c].at[pl.ds(s * SEG_STRIPE, SEG_STRIPE)])

    return k(e, row3, zeros)


@jax.jit
def _sc_k2(recip, row3):
    @functools.partial(
        pl.kernel,
        out_type=jax.ShapeDtypeStruct((N, HP), jnp.float32),
        mesh=_sc_mesh(),
        compiler_params=pltpu.CompilerParams(use_tc_tiling_on_sc=False),
        scratch_types=[
            pltpu.VMEM((CH, HP), jnp.float32),
            pltpu.VMEM((NCH, CW), jnp.int32),
            pltpu.VMEM_SHARED((NSEGP, HP), jnp.float32),
            pltpu.SemaphoreType.DMA,
        ],
    )
    def k(recip_hbm, row_hbm, r_hbm, g_v, row_v, recip_sh, sem):
        c = lax.axis_index("c")
        s = lax.axis_index("s")
        wid = s * 2 + c

        @pl.when(s == 0)
        def _():
            pltpu.sync_copy(recip_hbm, recip_sh)

        pltpu.async_copy(row_hbm.at[wid], row_v, sem).wait()
        plsc.subcore_barrier()

        @pl.loop(0, NCH)
        def _(j):
            pltpu.sync_copy(recip_sh.at[row_v.at[j]],
                            g_v.at[pl.ds(j * CW, CW)])

        pltpu.sync_copy(g_v, r_hbm.at[pl.ds(wid * CH, CH)])

    return k(recip, row3)


PHALF = NSEGP * HP // 128


def _tcmid_body(p_ref, o_ref):
    acc = p_ref[:PHALF, :] + p_ref[PHALF:, :]
    o_ref[...] = jnp.where(acc > 0.0, 1.0 / acc, 0.0)


def _tcmid(p_lin):
    return pl.pallas_call(
        _tcmid_body,
        in_specs=[pl.BlockSpec((2 * PHALF, 128), lambda: (0, 0))],
        out_specs=pl.BlockSpec((PHALF, 128), lambda: (0, 0)),
        out_shape=jax.ShapeDtypeStruct((PHALF, 128), jnp.float32),
    )(p_lin)


def _tc2_body(e_ref, r_ref, g_ref, o_ref):
    prod = e_ref[...] * r_ref[...]
    sums = 0.25 * jnp.dot(prod, g_ref[...],
                          preferred_element_type=jnp.float32)
    o_ref[...] = _pack_lanes(sums, HP)


def _tc2(e_lin, r_lin, g):
    grid = ((LIN + TB2L - 1) // TB2L,)
    return pl.pallas_call(
        _tc2_body,
        grid=grid,
        in_specs=[pl.BlockSpec((TB2L, 128), lambda i: (i, 0)),
                  pl.BlockSpec((TB2L, 128), lambda i: (i, 0)),
                  pl.BlockSpec((128, 128 // HP), lambda i: (0, 0))],
        out_specs=pl.BlockSpec((TB2L // HP, 128), lambda i: (i, 0)),
        out_shape=jax.ShapeDtypeStruct((OLIN, 128), jnp.float32),
    )(e_lin, r_lin, g)


def kernel(x, row, alpha, W1, b1, W2, b2, ln_g, ln_b, Wa, ba):
    b1_eff = (b1 + alpha[0, 0] * W1[:, D]).reshape(1, D).astype(jnp.bfloat16)
    w1t = W1[:, :D].T.astype(jnp.bfloat16)
    w2t = W2.T.astype(jnp.bfloat16)
    wa_g = Wa * ln_g[None, :]
    swa = jnp.sum(wa_g, axis=1)
    aug = (jnp.zeros((2 * D, HP), jnp.float32)
           .at[:D, :NH].set(wa_g.T - swa[None, :] * (1.0 / D))
           .at[:D, 8].set(1.0)
           .at[D:, 9].set(1.0)).astype(jnp.bfloat16)
    bb = ba + Wa @ ln_b
    lane = jnp.arange(128)
    l8 = (lane % HP == 8).astype(jnp.float32).reshape(1, 128)
    l9 = (lane % HP == 9).astype(jnp.float32).reshape(1, 128)
    bbp = jnp.where(lane % HP < NH, bb[lane % NH], 0.0).reshape(1, 128)
    maskp = (lane % HP < NH).astype(jnp.float32).reshape(1, 128)
    row3 = row.reshape(NW, NCH, CW)
    zeros = jnp.zeros((NSEGP, HP), jnp.float32)
    g = (jnp.arange(128)[:, None] // HP ==
         jnp.arange(128 // HP)[None, :]).astype(jnp.float32)

    e_lin = _tc1(x, w1t, w2t, aug, b1_eff,
                 b2.reshape(1, D).astype(jnp.bfloat16), l8, l9, bbp, maskp)
    e8 = e_lin.reshape(N, HP)
    partials = _sc_k1(e8, row3, zeros)
    recip_lin = _tcmid(partials.reshape(2 * PHALF, 128))
    r8 = _sc_k2(recip_lin.reshape(NSEGP, HP), row3)
    out_lin = _tc2(e_lin, r8.reshape(LIN, 128), g)
    return out_lin.reshape(N, 1)


# speedup vs baseline: 1.2712x; 1.0193x over previous
import functools

import jax
import jax.numpy as jnp
from jax import lax
from jax.experimental import pallas as pl
from jax.experimental.pallas import tpu as pltpu
from jax.experimental.pallas import tpu_sc as plsc

N = 160000
D = 256
NH = 4
HP = 16
NSEG = 10000
EPS = 1e-5

LIN = N * HP // 128
OLIN = N // 128

TB = 8000
TB2L = 2048

NW = 32
CH = N // NW
CW = 125
NCH = CH // CW
NSEGP = 10240
SEG_STRIPE = NSEGP // 16


def _pack_lanes(v, groups):
    r, k = v.shape
    v3 = v.reshape(r // groups, groups, k)
    return jnp.concatenate([v3[:, u, :] for u in range(groups)], axis=1)


def _tc1_body(x_ref, w1_ref, w2_ref, aug_ref, b1_ref, b2_ref,
              l8_ref, l9_ref, bbp_ref, maskp_ref, o_ref):
    HB = TB // 20
    HO = HB * HP // 128
    for half in range(20):
        xb = x_ref[half * HB:(half + 1) * HB, :].astype(jnp.bfloat16)
        acc1 = jnp.dot(xb, w1_ref[...], preferred_element_type=jnp.float32)
        h1b = jnp.maximum(acc1.astype(jnp.bfloat16) + b1_ref[...], 0)
        acc2 = jnp.dot(h1b, w2_ref[...], preferred_element_type=jnp.float32)
        h2b = jnp.maximum(acc2.astype(jnp.bfloat16) + b2_ref[...], 0)
        lhs = jnp.concatenate([h2b, h2b * h2b], axis=1)
        tm = jnp.dot(lhs, aug_ref[...], preferred_element_type=jnp.float32)
        tp = _pack_lanes(tm, 128 // HP)
        mu = tp * l8_ref[...] * (1.0 / D)
        s2l = jnp.roll(tp * l9_ref[...], -1, axis=1) * (1.0 / D)
        v = s2l - mu * mu
        v = v + jnp.roll(v, 1, axis=1)
        v = v + jnp.roll(v, 2, axis=1)
        v = v + jnp.roll(v, 4, axis=1)
        v = v + jnp.roll(v, -8, axis=1)
        inv = jax.lax.rsqrt(v + EPS)
        raw = inv * tp + bbp_ref[...]
        e = jnp.exp(jax.nn.sigmoid(raw)) * maskp_ref[...]
        o_ref[half * HO:(half + 1) * HO, :] = e


def _tc1(x, w1t, w2t, aug, b1r, b2r, l8, l9, bbp, maskp):
    grid = (N // TB,)
    full = lambda shape: pl.BlockSpec(shape, lambda i: (0, 0))
    return pl.pallas_call(
        _tc1_body,
        grid=grid,
        in_specs=[
            pl.BlockSpec((TB, D), lambda i: (i, 0)),
            full((D, D)), full((D, D)), full((2 * D, HP)),
            full((1, D)), full((1, D)),
            full((1, 128)), full((1, 128)), full((1, 128)), full((1, 128)),
        ],
        out_specs=pl.BlockSpec((TB * HP // 128, 128), lambda i: (i, 0)),
        out_shape=jax.ShapeDtypeStruct((LIN, 128), jnp.float32),
    )(x, w1t, w2t, aug, b1r, b2r, l8, l9, bbp, maskp)


def _sc_mesh():
    return plsc.VectorSubcoreMesh(core_axis_name="c", subcore_axis_name="s")


@jax.jit
def _sc_k1(e, row3, zeros):
    @functools.partial(
        pl.kernel,
        out_type=jax.ShapeDtypeStruct((2, NSEGP, HP), jnp.float32),
        mesh=_sc_mesh(),
        compiler_params=pltpu.CompilerParams(use_tc_tiling_on_sc=False),
        scratch_types=[
            pltpu.VMEM((CH, HP), jnp.float32),
            pltpu.VMEM((NCH, CW), jnp.int32),
            pltpu.VMEM_SHARED((NSEGP, HP), jnp.float32),
            pltpu.SemaphoreType.DMA,
        ],
    )
    def k(e_hbm, row_hbm, z_hbm, p_hbm, e_v, row_v, acc_sh, sem):
        c = lax.axis_index("c")
        s = lax.axis_index("s")
        wid = s * 2 + c

        @pl.when(s == 0)
        def _():
            pltpu.sync_copy(z_hbm, acc_sh)

        pltpu.async_copy(e_hbm.at[pl.ds(wid * CH, CH)], e_v, sem).wait()
        pltpu.async_copy(row_hbm.at[wid], row_v, sem).wait()
        plsc.subcore_barrier()

        @pl.loop(0, NCH)
        def _(j):
            pltpu.sync_copy(e_v.at[pl.ds(j * CW, CW)],
                            acc_sh.at[row_v.at[j]], add=True)

        plsc.subcore_barrier()
        pltpu.sync_copy(acc_sh.at[pl.ds(s * SEG_STRIPE, SEG_STRIPE)],
                        p_hbm.at[c].at[pl.ds(s * SEG_STRIPE, SEG_STRIPE)])

    return k(e, row3, zeros)


@jax.jit
def _sc_k2(recip, row3):
    @functools.partial(
        pl.kernel,
        out_type=jax.ShapeDtypeStruct((N, HP), jnp.float32),
        mesh=_sc_mesh(),
        compiler_params=pltpu.CompilerParams(use_tc_tiling_on_sc=False),
        scratch_types=[
            pltpu.VMEM((CH, HP), jnp.float32),
            pltpu.VMEM((NCH, CW), jnp.int32),
            pltpu.VMEM_SHARED((NSEGP, HP), jnp.float32),
            pltpu.SemaphoreType.DMA,
        ],
    )
    def k(recip_hbm, row_hbm, r_hbm, g_v, row_v, recip_sh, sem):
        c = lax.axis_index("c")
        s = lax.axis_index("s")
        wid = s * 2 + c

        @pl.when(s == 0)
        def _():
            pltpu.sync_copy(recip_hbm, recip_sh)

        pltpu.async_copy(row_hbm.at[wid], row_v, sem).wait()
        plsc.subcore_barrier()

        @pl.loop(0, NCH)
        def _(j):
            pltpu.sync_copy(recip_sh.at[row_v.at[j]],
                            g_v.at[pl.ds(j * CW, CW)])

        pltpu.sync_copy(g_v, r_hbm.at[pl.ds(wid * CH, CH)])

    return k(recip, row3)


PHALF = NSEGP * HP // 128


def _tcmid_body(p_ref, o_ref):
    acc = p_ref[:PHALF, :] + p_ref[PHALF:, :]
    o_ref[...] = jnp.where(acc > 0.0, 1.0 / acc, 0.0)


def _tcmid(p_lin):
    return pl.pallas_call(
        _tcmid_body,
        in_specs=[pl.BlockSpec((2 * PHALF, 128), lambda: (0, 0))],
        out_specs=pl.BlockSpec((PHALF, 128), lambda: (0, 0)),
        out_shape=jax.ShapeDtypeStruct((PHALF, 128), jnp.float32),
    )(p_lin)


def _tc2_body(e_ref, r_ref, g_ref, o_ref):
    prod = e_ref[...] * r_ref[...]
    sums = 0.25 * jnp.dot(prod, g_ref[...],
                          preferred_element_type=jnp.float32)
    o_ref[...] = _pack_lanes(sums, HP)


def _tc2(e_lin, r_lin, g):
    grid = ((LIN + TB2L - 1) // TB2L,)
    return pl.pallas_call(
        _tc2_body,
        grid=grid,
        in_specs=[pl.BlockSpec((TB2L, 128), lambda i: (i, 0)),
                  pl.BlockSpec((TB2L, 128), lambda i: (i, 0)),
                  pl.BlockSpec((128, 128 // HP), lambda i: (0, 0))],
        out_specs=pl.BlockSpec((TB2L // HP, 128), lambda i: (i, 0)),
        out_shape=jax.ShapeDtypeStruct((OLIN, 128), jnp.float32),
    )(e_lin, r_lin, g)


def kernel(x, row, alpha, W1, b1, W2, b2, ln_g, ln_b, Wa, ba):
    b1_eff = (b1 + alpha[0, 0] * W1[:, D]).reshape(1, D).astype(jnp.bfloat16)
    w1t = W1[:, :D].T.astype(jnp.bfloat16)
    w2t = W2.T.astype(jnp.bfloat16)
    wa_g = Wa * ln_g[None, :]
    swa = jnp.sum(wa_g, axis=1)
    aug = (jnp.zeros((2 * D, HP), jnp.float32)
           .at[:D, :NH].set(wa_g.T - swa[None, :] * (1.0 / D))
           .at[:D, 8].set(1.0)
           .at[D:, 9].set(1.0)).astype(jnp.bfloat16)
    bb = ba + Wa @ ln_b
    lane = jnp.arange(128)
    l8 = (lane % HP == 8).astype(jnp.float32).reshape(1, 128)
    l9 = (lane % HP == 9).astype(jnp.float32).reshape(1, 128)
    bbp = jnp.where(lane % HP < NH, bb[lane % NH], 0.0).reshape(1, 128)
    maskp = (lane % HP < NH).astype(jnp.float32).reshape(1, 128)
    row3 = row.reshape(NW, NCH, CW)
    zeros = jnp.zeros((NSEGP, HP), jnp.float32)
    g = (jnp.arange(128)[:, None] // HP ==
         jnp.arange(128 // HP)[None, :]).astype(jnp.float32)

    e_lin = _tc1(x, w1t, w2t, aug, b1_eff,
                 b2.reshape(1, D).astype(jnp.bfloat16), l8, l9, bbp, maskp)
    e8 = e_lin.reshape(N, HP)
    partials = _sc_k1(e8, row3, zeros)
    recip_lin = _tcmid(partials.reshape(2 * PHALF, 128))
    r8 = _sc_k2(recip_lin.reshape(NSEGP, HP), row3)
    out_lin = _tc2(e_lin, r8.reshape(LIN, 128), g)
    return out_lin.reshape(N, 1)
